# baseline (speedup 1.0000x reference)
import jax
import jax.numpy as jnp
from jax.experimental import pallas as pl
from jax.experimental.pallas import tpu as pltpu

_VOCAB = 128
_EMBED = 30
_F = 50
_C = 24
_NGRP = 2
_NCOL = 256
_W = 512


def _tables_kernel(emb_ref, wt_ref, t_ref):
    blocks = []
    for j in range(_NGRP):
        blocks.append(
            jax.lax.dot_general(
                emb_ref[...], wt_ref[j],
                dimension_numbers=(((1,), (0,)), ((), ())),
                preferred_element_type=jnp.float32,
                precision=jax.lax.Precision.HIGHEST,
            ))
    t_ref[...] = jnp.concatenate(blocks, axis=0).astype(jnp.bfloat16)


def _main_kernel(x_ref, t_ref, out_ref):
    lanes = _C * _W
    ids_t = jnp.transpose(x_ref[0])
    iota_v = jax.lax.broadcasted_iota(jnp.int32, (_VOCAB, _W), 0)
    chunks = []
    for p in range(_C):
        row = jnp.broadcast_to(ids_t[p:p + 1, :], (_VOCAB, _W))
        chunks.append(jnp.where(iota_v == row, 1.0, 0.0))
    g0 = jnp.concatenate(chunks, axis=1).astype(jnp.bfloat16)
    g1 = pltpu.roll(g0, lanes - _W, axis=1)
    lhs_t = jnp.concatenate([g0, g1], axis=0)
    y_t = jax.lax.dot_general(
        t_ref[...], lhs_t, dimension_numbers=(((0,), (0,)), ((), ())),
        preferred_element_type=jnp.float32)
    lr = pltpu.roll(y_t, lanes - 2 * _W, axis=1)
    z = y_t[0:104] + lr[104:208]
    base = z[:, :_W]
    for p in range(1, _C - 3):
        base = jnp.maximum(base, z[:, p * _W:(p + 1) * _W])
    m3 = jnp.maximum(base, z[:, (_C - 3) * _W:(_C - 2) * _W])
    yk2 = y_t[200:256]
    b2 = yk2[:, :_W]
    for p in range(1, _C - 1):
        b2 = jnp.maximum(b2, yk2[:, p * _W:(p + 1) * _W])
    pooled = jnp.concatenate(
        [b2[6:56], m3[0:_F], base[_F:2 * _F]], axis=0)
    out_ref[...] = jnp.transpose(jnp.maximum(pooled, 0.0))


@jax.jit
def kernel(x, emb_table, w2, b2, w3, b3, w4, b4):
    B, S, C = x.shape
    n_words = B * S
    n_blocks = n_words // _W

    ws = {2: w2, 3: w3, 4: w4}
    bs = {2: b2, 3: b3, 4: b4}

    def tap(k, j):
        return jnp.concatenate([ws[k][:, :, j].T, bs[k][None, :] / k], axis=0)

    z4 = jnp.zeros((_EMBED + 1, 4), jnp.float32)
    z2 = jnp.zeros((_EMBED + 1, 2), jnp.float32)
    z50 = jnp.zeros((_EMBED + 1, _F), jnp.float32)
    wt = jnp.stack([
        jnp.concatenate([tap(3, 0), tap(4, 0), z4, tap(3, 2), tap(4, 2),
                         z2, tap(2, 0)], axis=1),
        jnp.concatenate([tap(3, 1), tap(4, 1), z4, z50, tap(4, 3),
                         z2, tap(2, 1)], axis=1),
    ])
    emb_ext = jnp.pad(emb_table, ((0, 0), (0, 1)), constant_values=1.0)

    t_cat = pl.pallas_call(
        _tables_kernel,
        out_shape=jax.ShapeDtypeStruct((_NGRP * _VOCAB, _NCOL), jnp.bfloat16),
    )(emb_ext, wt)

    x_blk = x.reshape(n_blocks, _W, _C)

    out = pl.pallas_call(
        _main_kernel,
        grid=(n_blocks,),
        in_specs=[
            pl.BlockSpec((1, _W, _C), lambda i: (i, 0, 0)),
            pl.BlockSpec((_NGRP * _VOCAB, _NCOL), lambda i: (0, 0)),
        ],
        out_specs=pl.BlockSpec((_W, 3 * _F), lambda i: (i, 0)),
        out_shape=jax.ShapeDtypeStruct((n_words, 3 * _F), jnp.float32),
        compiler_params=pltpu.CompilerParams(
            dimension_semantics=("parallel",)),
    )(x_blk, t_cat)

    return out.reshape(B, S, 3 * _F)


# speedup vs baseline: 1.0883x; 1.0883x over previous
import jax
import jax.numpy as jnp
from jax.experimental import pallas as pl
from jax.experimental.pallas import tpu as pltpu

_VOCAB = 128
_EMBED = 30
_F = 50
_C = 24
_NGRP = 2
_NCOL = 256
_W = 1024


def _tables_kernel(emb_ref, wt_ref, t_ref):
    blocks = []
    for j in range(_NGRP):
        blocks.append(
            jax.lax.dot_general(
                emb_ref[...], wt_ref[j],
                dimension_numbers=(((1,), (0,)), ((), ())),
                preferred_element_type=jnp.float32,
                precision=jax.lax.Precision.HIGHEST,
            ))
    t_ref[...] = jnp.concatenate(blocks, axis=0)


def _main_kernel(x_ref, t_ref, out_ref):
    lanes = _C * _W
    ids_t = jnp.transpose(x_ref[0])
    iota_v = jax.lax.broadcasted_iota(jnp.int32, (_VOCAB, _W), 0)
    chunks = []
    for p in range(_C):
        row = jnp.broadcast_to(ids_t[p:p + 1, :], (_VOCAB, _W))
        chunks.append(jnp.where(iota_v == row, 1.0, 0.0))
    g0 = jnp.concatenate(chunks, axis=1)
    g1 = pltpu.roll(g0, lanes - _W, axis=1)
    lhs_t = jnp.concatenate([g0, g1], axis=0)
    y_t = jax.lax.dot_general(
        t_ref[...], lhs_t, dimension_numbers=(((0,), (0,)), ((), ())),
        preferred_element_type=jnp.float32)
    lr = pltpu.roll(y_t, lanes - 2 * _W, axis=1)
    z = y_t[0:104] + lr[104:208]
    base = z[:, :_W]
    for p in range(1, _C - 3):
        base = jnp.maximum(base, z[:, p * _W:(p + 1) * _W])
    m3 = jnp.maximum(base, z[:, (_C - 3) * _W:(_C - 2) * _W])
    yk2 = y_t[200:256]
    b2 = yk2[:, :_W]
    for p in range(1, _C - 1):
        b2 = jnp.maximum(b2, yk2[:, p * _W:(p + 1) * _W])
    pooled = jnp.concatenate(
        [b2[6:56], m3[0:_F], base[_F:2 * _F]], axis=0)
    out_ref[...] = jnp.transpose(jnp.maximum(pooled, 0.0))


@jax.jit
def kernel(x, emb_table, w2, b2, w3, b3, w4, b4):
    B, S, C = x.shape
    n_words = B * S
    n_blocks = n_words // _W

    ws = {2: w2, 3: w3, 4: w4}
    bs = {2: b2, 3: b3, 4: b4}

    def tap(k, j):
        return jnp.concatenate([ws[k][:, :, j].T, bs[k][None, :] / k], axis=0)

    z4 = jnp.zeros((_EMBED + 1, 4), jnp.float32)
    z2 = jnp.zeros((_EMBED + 1, 2), jnp.float32)
    z50 = jnp.zeros((_EMBED + 1, _F), jnp.float32)
    wt = jnp.stack([
        jnp.concatenate([tap(3, 0), tap(4, 0), z4, tap(3, 2), tap(4, 2),
                         z2, tap(2, 0)], axis=1),
        jnp.concatenate([tap(3, 1), tap(4, 1), z4, z50, tap(4, 3),
                         z2, tap(2, 1)], axis=1),
    ])
    emb_ext = jnp.pad(emb_table, ((0, 0), (0, 1)), constant_values=1.0)

    t_cat = pl.pallas_call(
        _tables_kernel,
        out_shape=jax.ShapeDtypeStruct((_NGRP * _VOCAB, _NCOL), jnp.float32),
    )(emb_ext, wt)

    x_blk = x.reshape(n_blocks, _W, _C)

    out = pl.pallas_call(
        _main_kernel,
        grid=(n_blocks,),
        in_specs=[
            pl.BlockSpec((1, _W, _C), lambda i: (i, 0, 0)),
            pl.BlockSpec((_NGRP * _VOCAB, _NCOL), lambda i: (0, 0)),
        ],
        out_specs=pl.BlockSpec((_W, 3 * _F), lambda i: (i, 0)),
        out_shape=jax.ShapeDtypeStruct((n_words, 3 * _F), jnp.float32),
        compiler_params=pltpu.CompilerParams(
            dimension_semantics=("parallel",)),
    )(x_blk, t_cat)

    return out.reshape(B, S, 3 * _F)
